# baseline (speedup 1.0000x reference)
import jax
import jax.numpy as jnp
from jax.experimental import pallas as pl
from jax.experimental.pallas import tpu as pltpu

_N_RF = 8


def _fused_body(xyz_ref, rots_ref, offs_ref, W1_ref, b1_ref, Waw_ref,
                sigma_ref, feat_ref):
    xb = xyz_ref[...].astype(jnp.bfloat16)
    best_clip = None
    sigma = None
    feat = None
    for r in range(_N_RF):
        rxyz = jax.lax.dot_general(
            xb, rots_ref[r], (((1,), (1,)), ((), ())),
            preferred_element_type=jnp.float32)
        oxyz = (rxyz + offs_ref[r]).astype(jnp.bfloat16)
        pre = jax.lax.dot_general(
            oxyz, W1_ref[r], (((1,), (0,)), ((), ())),
            preferred_element_type=jnp.float32)
        h = jnp.maximum(pre + b1_ref[r][None, :], 0.0)
        out = jax.lax.dot_general(
            h.astype(jnp.bfloat16), Waw_ref[r], (((1,), (0,)), ((), ())),
            preferred_element_type=jnp.float32)
        ft = out[:, :128].astype(jnp.bfloat16)
        sig = out[:, 128:129]
        clip = jnp.clip(sig, -10.0, 10.0)
        if r == 0:
            best_clip, sigma, feat = clip, sig, ft
        else:
            upd = clip > best_clip
            best_clip = jnp.where(upd, clip, best_clip)
            sigma = jnp.where(upd, sig, sigma)
            feat = jnp.where(upd, ft, feat)
    sigma_ref[...] = sigma
    feat_ref[...] = feat


def kernel(xyz, rots, offsets, aabbs, W1, b1, wd, Wa):
    del aabbs
    n = xyz.shape[0]
    blk = 1024
    grid = (n // blk,)
    Waw = jnp.concatenate(
        [Wa, wd[:, :, None], jnp.zeros((_N_RF, 256, 127), jnp.float32)],
        axis=2).astype(jnp.bfloat16)
    whole = lambda *dims: pl.BlockSpec(dims, lambda i: (0,) * len(dims))
    sigma2, feat = pl.pallas_call(
        _fused_body,
        grid=grid,
        in_specs=[
            pl.BlockSpec((blk, 3), lambda i: (i, 0)),
            whole(_N_RF, 3, 3),
            whole(_N_RF, 1, 3),
            whole(_N_RF, 3, 256),
            whole(_N_RF, 256),
            whole(_N_RF, 256, 256),
        ],
        out_specs=[
            pl.BlockSpec((blk, 1), lambda i: (i, 0)),
            pl.BlockSpec((blk, 128), lambda i: (i, 0)),
        ],
        out_shape=[
            jax.ShapeDtypeStruct((n, 1), jnp.float32),
            jax.ShapeDtypeStruct((n, 128), jnp.bfloat16),
        ],
        compiler_params=pltpu.CompilerParams(
            dimension_semantics=("parallel",)),
    )(xyz, rots.astype(jnp.bfloat16), offsets[:, :1, :3],
      W1.astype(jnp.bfloat16), b1, Waw)
    return sigma2.reshape(-1), feat.astype(jnp.float32)


# speedup vs baseline: 1.3582x; 1.3582x over previous
import jax
import jax.numpy as jnp
from jax.experimental import pallas as pl
from jax.experimental.pallas import tpu as pltpu

_N_RF = 8


def _fused_body(xyz_ref, rt_ref, off_ref, W1e_ref, Waw_ref,
                sigma_ref, feat_ref):
    blk = xyz_ref.shape[0]
    xb = xyz_ref[...].astype(jnp.bfloat16)
    rx = jax.lax.dot_general(
        xb, rt_ref[...], (((1,), (0,)), ((), ())),
        preferred_element_type=jnp.float32)
    ox = (rx + off_ref[...]).astype(jnp.bfloat16)
    X = jnp.concatenate(
        [ox, jnp.ones((blk, 3), jnp.bfloat16)], axis=1)
    best_clip = None
    sigma = None
    feat = None
    for r in range(_N_RF):
        pre = jax.lax.dot_general(
            X, W1e_ref[r], (((1,), (0,)), ((), ())),
            preferred_element_type=jnp.float32)
        h = jnp.maximum(pre, 0.0).astype(jnp.bfloat16)
        out = jax.lax.dot_general(
            h, Waw_ref[r], (((1,), (0,)), ((), ())),
            preferred_element_type=jnp.float32)
        ft = out[:, :128]
        sig = out[:, 128:129]
        clip = jnp.clip(sig, -10.0, 10.0)
        if r == 0:
            best_clip, sigma, feat = clip, sig, ft
        else:
            upd = clip > best_clip
            best_clip = jnp.where(upd, clip, best_clip)
            sigma = jnp.where(upd, sig, sigma)
            feat = jnp.where(upd, ft, feat)
    sigma_ref[...] = sigma
    feat_ref[...] = feat


def kernel(xyz, rots, offsets, aabbs, W1, b1, wd, Wa):
    del aabbs
    n = xyz.shape[0]
    blk = 1024
    grid = (n // blk,)
    f32 = jnp.float32
    bf16 = jnp.bfloat16
    rt = jnp.transpose(rots, (2, 0, 1)).reshape(3, 24).astype(bf16)
    offflat = offsets[:, 0, :3].reshape(1, 24).astype(f32)
    b1_hi = b1.astype(bf16)
    r1 = b1 - b1_hi.astype(f32)
    b1_mid = r1.astype(bf16)
    b1_lo = (r1 - b1_mid.astype(f32)).astype(bf16)
    W1e = jnp.zeros((_N_RF, 27, 256), f32)
    for r in range(_N_RF):
        W1e = W1e.at[r, 3 * r:3 * r + 3, :].set(
            W1[r].astype(bf16).astype(f32))
    W1e = W1e.at[:, 24, :].set(b1_hi.astype(f32))
    W1e = W1e.at[:, 25, :].set(b1_mid.astype(f32))
    W1e = W1e.at[:, 26, :].set(b1_lo.astype(f32))
    W1e = W1e.astype(bf16)
    Waw = jnp.concatenate(
        [Wa, wd[:, :, None], jnp.zeros((_N_RF, 256, 127), f32)],
        axis=2).astype(bf16)
    whole = lambda *dims: pl.BlockSpec(dims, lambda i: (0,) * len(dims))
    sigma2, feat = pl.pallas_call(
        _fused_body,
        grid=grid,
        in_specs=[
            pl.BlockSpec((blk, 3), lambda i: (i, 0)),
            whole(3, 24),
            whole(1, 24),
            whole(_N_RF, 27, 256),
            whole(_N_RF, 256, 256),
        ],
        out_specs=[
            pl.BlockSpec((blk, 1), lambda i: (i, 0)),
            pl.BlockSpec((blk, 128), lambda i: (i, 0)),
        ],
        out_shape=[
            jax.ShapeDtypeStruct((n, 1), f32),
            jax.ShapeDtypeStruct((n, 128), f32),
        ],
        compiler_params=pltpu.CompilerParams(
            dimension_semantics=("parallel",)),
    )(xyz, rt, offflat, W1e, Waw)
    return sigma2.reshape(-1), feat
